# baseline (speedup 1.0000x reference)
import functools

import jax
import jax.numpy as jnp
from jax import lax
from jax.experimental import pallas as pl
from jax.experimental.pallas import tpu as pltpu
from jax.experimental.pallas import tpu_sc as plsc

EMBED = 64
ROW = 128
NUM_CORES = 2
NUM_SUBCORES = 16
NUM_WORKERS = NUM_CORES * NUM_SUBCORES
CHUNK = 200
NBUF = 4


@functools.lru_cache(maxsize=None)
def _build(n_rows):
    per_w = n_rows // NUM_WORKERS
    n_chunks = per_w // CHUNK
    n_groups = n_chunks // NBUF
    mesh = plsc.VectorSubcoreMesh(core_axis_name="c", subcore_axis_name="s")

    scratch = (
        [pltpu.VMEM((CHUNK,), jnp.int32) for _ in range(NBUF)]
        + [pltpu.VMEM((CHUNK, ROW), jnp.float32) for _ in range(NBUF)]
        + [pltpu.SemaphoreType.DMA for _ in range(3 * NBUF)]
    )

    @functools.partial(
        pl.kernel,
        out_type=jax.ShapeDtypeStruct((n_rows, ROW), jnp.float32),
        mesh=mesh,
        scratch_types=scratch,
        compiler_params=pltpu.CompilerParams(use_tc_tiling_on_sc=True),
    )
    def gather_kernel(idx_hbm, table_hbm, out_hbm, *refs):
        idxb = refs[:NBUF]
        rows = refs[NBUF : 2 * NBUF]
        isem = refs[2 * NBUF : 3 * NBUF]
        gsem = refs[3 * NBUF : 4 * NBUF]
        ssem = refs[4 * NBUF :]
        wid = lax.axis_index("s") * NUM_CORES + lax.axis_index("c")
        base = wid * per_w

        def start_idx(c, b):
            return pltpu.async_copy(
                idx_hbm.at[pl.ds(base + c * CHUNK, CHUNK)], idxb[b], isem[b]
            )

        def wait_idx(b):
            pltpu.make_async_copy(
                idx_hbm.at[pl.ds(base, CHUNK)], idxb[b], isem[b]
            ).wait()

        def start_gather(b):
            return pltpu.async_copy(table_hbm.at[idxb[b]], rows[b], gsem[b])

        def start_scatter(c, b):
            return pltpu.async_copy(
                rows[b], out_hbm.at[pl.ds(base + c * CHUNK, CHUNK)], ssem[b]
            )

        def wait_scatter(b):
            pltpu.make_async_copy(
                rows[b], out_hbm.at[pl.ds(base, CHUNK)], ssem[b]
            ).wait()

        for j in range(NBUF):
            start_idx(j, j)

        def group(g, carry):
            gds = []
            for j in range(NBUF):
                @pl.when(g >= 1)
                def _():
                    wait_scatter(j)

                wait_idx(j)
                gds.append(start_gather(j))
            for j in range(NBUF):
                c = g * NBUF + j
                gds[j].wait()
                start_scatter(c, j)

                @pl.when(g <= n_groups - 2)
                def _():
                    start_idx(c + NBUF, j)

            return carry

        lax.fori_loop(0, n_groups, group, 0)
        for j in range(NBUF):
            wait_scatter(j)

    return gather_kernel


@functools.lru_cache(maxsize=None)
def _build_fmt(vocab):
    blk = 8192
    nblk = (vocab + blk - 1) // blk

    def fmt_kernel(t_ref, o_ref):
        eye = jnp.eye(EMBED, ROW, dtype=jnp.float32)
        o_ref[...] = jax.lax.dot_general(
            t_ref[...], eye, (((0,), (0,)), ((), ())),
            preferred_element_type=jnp.float32,
            precision=jax.lax.Precision.HIGHEST,
        )

    return pl.pallas_call(
        fmt_kernel,
        grid=(nblk,),
        in_specs=[pl.BlockSpec((EMBED, blk), lambda i: (0, i))],
        out_specs=pl.BlockSpec((blk, ROW), lambda i: (i, 0)),
        out_shape=jax.ShapeDtypeStruct((vocab, ROW), jnp.float32),
    )


def kernel(sentences, table):
    b, l = sentences.shape
    v = table.shape[0]
    idx = sentences.reshape(b * l)
    table128 = _build_fmt(v)(table.T)
    out = _build(b * l)(idx, table128)
    return out[:, :EMBED].reshape(b, l, EMBED)


# speedup vs baseline: 1.2560x; 1.2560x over previous
import functools

import jax
import jax.numpy as jnp
from jax import lax
from jax.experimental import pallas as pl
from jax.experimental.pallas import tpu as pltpu
from jax.experimental.pallas import tpu_sc as plsc

EMBED = 64
ROW = 128
NUM_CORES = 2
NUM_SUBCORES = 16
NUM_WORKERS = NUM_CORES * NUM_SUBCORES
CHUNK = 200
NBUF = 4


@functools.lru_cache(maxsize=None)
def _build(n_rows):
    per_w = n_rows // NUM_WORKERS
    n_chunks = per_w // CHUNK
    n_groups = n_chunks // NBUF
    mesh = plsc.VectorSubcoreMesh(core_axis_name="c", subcore_axis_name="s")

    scratch = (
        [pltpu.VMEM((CHUNK,), jnp.int32) for _ in range(NBUF)]
        + [pltpu.VMEM((CHUNK, ROW), jnp.float32) for _ in range(NBUF)]
        + [pltpu.SemaphoreType.DMA for _ in range(3 * NBUF)]
    )

    @functools.partial(
        pl.kernel,
        out_type=jax.ShapeDtypeStruct((n_rows, ROW), jnp.float32),
        mesh=mesh,
        scratch_types=scratch,
        compiler_params=pltpu.CompilerParams(use_tc_tiling_on_sc=True),
    )
    def gather_kernel(idx_hbm, table_hbm, out_hbm, *refs):
        idxb = refs[:NBUF]
        rows = refs[NBUF : 2 * NBUF]
        isem = refs[2 * NBUF : 3 * NBUF]
        gsem = refs[3 * NBUF : 4 * NBUF]
        ssem = refs[4 * NBUF :]
        wid = lax.axis_index("s") * NUM_CORES + lax.axis_index("c")
        base = wid * per_w

        def start_idx(c, b):
            return pltpu.async_copy(
                idx_hbm.at[pl.ds(base + c * CHUNK, CHUNK)], idxb[b], isem[b]
            )

        def wait_idx(b):
            pltpu.make_async_copy(
                idx_hbm.at[pl.ds(base, CHUNK)], idxb[b], isem[b]
            ).wait()

        def start_gather(b):
            return pltpu.async_copy(table_hbm.at[idxb[b]], rows[b], gsem[b])

        def start_scatter(c, b):
            return pltpu.async_copy(
                rows[b], out_hbm.at[pl.ds(base + c * CHUNK, CHUNK)], ssem[b]
            )

        def wait_scatter(b):
            pltpu.make_async_copy(
                rows[b], out_hbm.at[pl.ds(base, CHUNK)], ssem[b]
            ).wait()

        for j in range(NBUF):
            start_idx(j, j)

        def group(g, carry):
            gds = []
            for j in range(NBUF):
                @pl.when(g >= 1)
                def _():
                    wait_scatter(j)

                wait_idx(j)
                gds.append(start_gather(j))
            for j in range(NBUF):
                c = g * NBUF + j
                gds[j].wait()
                start_scatter(c, j)

                @pl.when(g <= n_groups - 2)
                def _():
                    start_idx(c + NBUF, j)

            return carry

        lax.fori_loop(0, n_groups, group, 0)
        for j in range(NBUF):
            wait_scatter(j)

    return gather_kernel


@functools.lru_cache(maxsize=None)
def _build_fmt(vocab):
    blk = 16384
    nblk = (vocab + blk - 1) // blk

    def fmt_kernel(t_ref, o_ref):
        t = t_ref[...].T
        o_ref[...] = jnp.concatenate(
            [t, jnp.zeros(t.shape, jnp.float32)], axis=1
        )

    return pl.pallas_call(
        fmt_kernel,
        grid=(nblk,),
        in_specs=[pl.BlockSpec((EMBED, blk), lambda i: (0, i))],
        out_specs=pl.BlockSpec((blk, ROW), lambda i: (i, 0)),
        out_shape=jax.ShapeDtypeStruct((vocab, ROW), jnp.float32),
    )


def kernel(sentences, table):
    b, l = sentences.shape
    v = table.shape[0]
    idx = sentences.reshape(b * l)
    table128 = _build_fmt(v)(table.T)
    out = _build(b * l)(idx, table128)
    return out[:, :EMBED].reshape(b, l, EMBED)


# speedup vs baseline: 1.5700x; 1.2500x over previous
import functools

import jax
import jax.numpy as jnp
from jax import lax
from jax.experimental import pallas as pl
from jax.experimental.pallas import tpu as pltpu
from jax.experimental.pallas import tpu_sc as plsc

EMBED = 64
ROW = 128
NUM_CORES = 2
NUM_SUBCORES = 16
NUM_WORKERS = NUM_CORES * NUM_SUBCORES
CHUNK = 200
NBUF = 4


@functools.lru_cache(maxsize=None)
def _build(n_rows):
    per_w = n_rows // NUM_WORKERS
    n_chunks = per_w // CHUNK
    n_groups = n_chunks // NBUF
    mesh = plsc.VectorSubcoreMesh(core_axis_name="c", subcore_axis_name="s")

    scratch = (
        [pltpu.VMEM((CHUNK,), jnp.int32) for _ in range(NBUF)]
        + [pltpu.VMEM((CHUNK, EMBED), jnp.float32) for _ in range(NBUF)]
        + [pltpu.SemaphoreType.DMA for _ in range(3 * NBUF)]
    )

    @functools.partial(
        pl.kernel,
        out_type=jax.ShapeDtypeStruct((n_rows, ROW), jnp.float32),
        mesh=mesh,
        scratch_types=scratch,
        compiler_params=pltpu.CompilerParams(use_tc_tiling_on_sc=False),
    )
    def gather_kernel(idx_hbm, table_hbm, out_hbm, *refs):
        idxb = refs[:NBUF]
        rows = refs[NBUF : 2 * NBUF]
        isem = refs[2 * NBUF : 3 * NBUF]
        gsem = refs[3 * NBUF : 4 * NBUF]
        ssem = refs[4 * NBUF :]
        wid = lax.axis_index("s") * NUM_CORES + lax.axis_index("c")
        base = wid * per_w

        def start_idx(c, b):
            return pltpu.async_copy(
                idx_hbm.at[pl.ds(base + c * CHUNK, CHUNK)], idxb[b], isem[b]
            )

        def wait_idx(b):
            pltpu.make_async_copy(
                idx_hbm.at[pl.ds(base, CHUNK)], idxb[b], isem[b]
            ).wait()

        def start_gather(b):
            return pltpu.async_copy(table_hbm.at[idxb[b]], rows[b], gsem[b])

        def start_scatter(c, b):
            return pltpu.async_copy(
                rows[b],
                out_hbm.at[pl.ds(base + c * CHUNK, CHUNK), pl.ds(0, EMBED)],
                ssem[b],
            )

        def wait_scatter(b):
            pltpu.make_async_copy(
                rows[b],
                out_hbm.at[pl.ds(base, CHUNK), pl.ds(0, EMBED)],
                ssem[b],
            ).wait()

        for j in range(NBUF):
            start_idx(j, j)

        def group(g, carry):
            gds = []
            for j in range(NBUF):
                @pl.when(g >= 1)
                def _():
                    wait_scatter(j)

                wait_idx(j)
                gds.append(start_gather(j))
            for j in range(NBUF):
                c = g * NBUF + j
                gds[j].wait()
                start_scatter(c, j)

                @pl.when(g <= n_groups - 2)
                def _():
                    start_idx(c + NBUF, j)

            return carry

        lax.fori_loop(0, n_groups, group, 0)
        for j in range(NBUF):
            wait_scatter(j)

    return gather_kernel


@functools.lru_cache(maxsize=None)
def _build_fmt(vocab):
    blk = 16384
    nblk = (vocab + blk - 1) // blk

    def fmt_kernel(t_ref, o_ref):
        t = t_ref[...].T
        o_ref[...] = jnp.concatenate(
            [t, jnp.zeros(t.shape, jnp.float32)], axis=1
        )

    return pl.pallas_call(
        fmt_kernel,
        grid=(nblk,),
        in_specs=[pl.BlockSpec((EMBED, blk), lambda i: (0, i))],
        out_specs=pl.BlockSpec((blk, ROW), lambda i: (i, 0)),
        out_shape=jax.ShapeDtypeStruct((vocab, ROW), jnp.float32),
    )


def kernel(sentences, table):
    b, l = sentences.shape
    v = table.shape[0]
    idx = sentences.reshape(b * l) * 2
    table128 = _build_fmt(v)(table.T)
    table2m = table128.reshape(2 * v, EMBED)
    out = _build(b * l)(idx, table2m)
    return out[:, :EMBED].reshape(b, l, EMBED)


# speedup vs baseline: 1.5876x; 1.0112x over previous
import functools

import jax
import jax.numpy as jnp
from jax import lax
from jax.experimental import pallas as pl
from jax.experimental.pallas import tpu as pltpu
from jax.experimental.pallas import tpu_sc as plsc

EMBED = 64
ROW = 128
NUM_CORES = 2
NUM_SUBCORES = 16
NUM_WORKERS = NUM_CORES * NUM_SUBCORES
CHUNK = 200
NBUF = 4


@functools.lru_cache(maxsize=None)
def _build(n_rows):
    per_w = n_rows // NUM_WORKERS
    n_chunks = per_w // CHUNK
    n_groups = n_chunks // NBUF
    mesh = plsc.VectorSubcoreMesh(core_axis_name="c", subcore_axis_name="s")

    scratch = (
        [pltpu.VMEM((CHUNK,), jnp.int32) for _ in range(NBUF)]
        + [pltpu.VMEM((CHUNK, EMBED), jnp.float32) for _ in range(NBUF)]
        + [pltpu.SemaphoreType.DMA for _ in range(3 * NBUF)]
    )

    @functools.partial(
        pl.kernel,
        out_type=jax.ShapeDtypeStruct((n_rows, ROW), jnp.float32),
        mesh=mesh,
        scratch_types=scratch,
        compiler_params=pltpu.CompilerParams(use_tc_tiling_on_sc=False),
    )
    def gather_kernel(idx_hbm, table_hbm, out_hbm, *refs):
        idxb = refs[:NBUF]
        rows = refs[NBUF : 2 * NBUF]
        isem = refs[2 * NBUF : 3 * NBUF]
        gsem = refs[3 * NBUF : 4 * NBUF]
        ssem = refs[4 * NBUF :]
        wid = lax.axis_index("s") * NUM_CORES + lax.axis_index("c")
        base = wid * per_w

        def start_idx(c, b):
            return pltpu.async_copy(
                idx_hbm.at[pl.ds(base + c * CHUNK, CHUNK)], idxb[b], isem[b]
            )

        def wait_idx(b):
            pltpu.make_async_copy(
                idx_hbm.at[pl.ds(base, CHUNK)], idxb[b], isem[b]
            ).wait()

        def start_gather(b):
            return pltpu.async_copy(table_hbm.at[idxb[b]], rows[b], gsem[b])

        def start_scatter(c, b):
            return pltpu.async_copy(
                rows[b],
                out_hbm.at[pl.ds(base + c * CHUNK, CHUNK), pl.ds(0, EMBED)],
                ssem[b],
            )

        def wait_scatter(b):
            pltpu.make_async_copy(
                rows[b],
                out_hbm.at[pl.ds(base, CHUNK), pl.ds(0, EMBED)],
                ssem[b],
            ).wait()

        for j in range(NBUF):
            start_idx(j, j)

        def group(g, carry):
            gds = []
            for j in range(NBUF):
                @pl.when(g >= 1)
                def _():
                    wait_scatter(j)

                wait_idx(j)
                gds.append(start_gather(j))
            for j in range(NBUF):
                c = g * NBUF + j
                gds[j].wait()
                start_scatter(c, j)

                @pl.when(g <= n_groups - 2)
                def _():
                    start_idx(c + NBUF, j)

            return carry

        lax.fori_loop(0, n_groups, group, 0)
        for j in range(NBUF):
            wait_scatter(j)

    return gather_kernel


@functools.lru_cache(maxsize=None)
def _build_fmt(vocab):
    blk = 32768
    nblk = (vocab + blk - 1) // blk

    def fmt_kernel(t_ref, o_ref):
        t = t_ref[...].T
        o_ref[...] = jnp.concatenate(
            [t, jnp.zeros(t.shape, jnp.float32)], axis=1
        )

    return pl.pallas_call(
        fmt_kernel,
        grid=(nblk,),
        in_specs=[pl.BlockSpec((EMBED, blk), lambda i: (0, i))],
        out_specs=pl.BlockSpec((blk, ROW), lambda i: (i, 0)),
        out_shape=jax.ShapeDtypeStruct((vocab, ROW), jnp.float32),
    )


def kernel(sentences, table):
    b, l = sentences.shape
    v = table.shape[0]
    idx = sentences.reshape(b * l) * 2
    table128 = _build_fmt(v)(table.T)
    table2m = table128.reshape(2 * v, EMBED)
    out = _build(b * l)(idx, table2m)
    return out[:, :EMBED].reshape(b, l, EMBED)
